# Initial kernel scaffold; baseline (speedup 1.0000x reference)
import jax
import jax.numpy as jnp
from jax.experimental import pallas as pl


def kernel(inp, rotations):
    raise NotImplementedError("write your pallas kernel here")


# speedup vs baseline: 29.5602x; 29.5602x over previous
import jax
import jax.numpy as jnp
from jax.experimental import pallas as pl

_WS = 32
_N_HASHES = 4
_HALF = 4

_COLOR_R = (0, 46, 167, 100, 191, 220, 0, 10)
_COLOR_G = (160, 141, 0, 62, 30, 87, 166, 91)
_COLOR_B = (177, 239, 174, 191, 75, 46, 0, 196)


def _hash_kernel(x_ref, w_ref, r_ref, g_ref, b_ref):
    x = x_ref[0]
    w = w_ref[...]
    v = jax.lax.dot_general(w, x, (((0,), (0,)), ((), ())),
                            preferred_element_type=jnp.float32)
    codes = []
    for h in range(_N_HASHES):
        vh = v[h * _HALF:(h + 1) * _HALF]
        full = jnp.concatenate([vh, -vh], axis=0)
        codes.append(jnp.argmax(full, axis=0, keepdims=True).astype(jnp.int32))
    code = jnp.concatenate(codes, axis=0)
    for ref, table in ((r_ref, _COLOR_R), (g_ref, _COLOR_G), (b_ref, _COLOR_B)):
        acc = jnp.zeros(code.shape, jnp.int32)
        for k, val in enumerate(table):
            if val:
                acc = acc + jnp.where(code == k, jnp.int32(val), jnp.int32(0))
        ref[0] = acc.astype(jnp.uint8)


def kernel(inp, rotations):
    B, C, H, W = inp.shape
    ws = _WS
    nH, nW = H // ws, W // ws
    M = nH * nW
    T = ws * ws
    NC = B * M

    E4 = (inp.reshape(B, C, nH, ws, nW, ws)
             .transpose(0, 1, 2, 4, 3, 5)
             .reshape(NC, C, T))
    w2 = rotations.reshape(C, _N_HASHES * _HALF)

    out_sds = [jax.ShapeDtypeStruct((NC, _N_HASHES, T), jnp.uint8)
               for _ in range(3)]
    r, g, b = pl.pallas_call(
        _hash_kernel,
        grid=(NC,),
        in_specs=[
            pl.BlockSpec((1, C, T), lambda n: (n, 0, 0)),
            pl.BlockSpec((C, _N_HASHES * _HALF), lambda n: (0, 0)),
        ],
        out_specs=[pl.BlockSpec((1, _N_HASHES, T), lambda n: (n, 0, 0))
                   for _ in range(3)],
        out_shape=out_sds,
    )(E4, w2)

    def fin(a):
        a6 = a.reshape(B, _N_HASHES, nH, nW, ws, ws)
        return a6.transpose(0, 2, 4, 3, 5, 1).reshape(B, H, W, _N_HASHES)

    return fin(r), fin(g), fin(b)


# speedup vs baseline: 32.9867x; 1.1159x over previous
import math

import jax
import jax.numpy as jnp
from jax.experimental import pallas as pl

_WS = 32
_N_HASHES = 4
_HALF = 4

_COLOR_R = (0, 46, 167, 100, 191, 220, 0, 10)
_COLOR_G = (160, 141, 0, 62, 30, 87, 166, 91)
_COLOR_B = (177, 239, 174, 191, 75, 46, 0, 196)


def _make_hash_kernel(chg, nH, nW, ws, n_chunks):
    T = ws * ws

    def _hash_kernel(x_ref, w_ref, r_ref, g_ref, b_ref):
        x = x_ref[0]
        xe = x.transpose(0, 1, 3, 2, 4).reshape(chg * nH * nW, T)
        w = w_ref[...]
        v = jax.lax.dot_general(w, xe, (((0,), (0,)), ((), ())),
                                preferred_element_type=jnp.float32)
        codes = []
        for q in range(n_chunks):
            for h in range(_N_HASHES):
                vh = v[q * 16 + h * _HALF:q * 16 + (h + 1) * _HALF]
                full = jnp.concatenate([vh, -vh], axis=0)
                codes.append(
                    jnp.argmax(full, axis=0, keepdims=True).astype(jnp.int32))
        code = jnp.concatenate(codes, axis=0)
        for ref, table in ((r_ref, _COLOR_R), (g_ref, _COLOR_G),
                           (b_ref, _COLOR_B)):
            acc = jnp.zeros(code.shape, jnp.int32)
            for k, val in enumerate(table):
                if val:
                    acc = acc + jnp.where(code == k, jnp.int32(val),
                                          jnp.int32(0))
            ref[0] = acc.astype(jnp.uint8)

    return _hash_kernel


def kernel(inp, rotations):
    B, C, H, W = inp.shape
    ws = _WS
    nH, nW = H // ws, W // ws
    M = nH * nW
    T = ws * ws
    R = math.lcm(C, M)
    chg = R // M
    n_chunks = R // C
    n_groups = (B * C) // chg

    x6 = inp.reshape(n_groups, chg, nH, ws, nW, ws)
    w2 = rotations.reshape(C, _N_HASHES * _HALF)
    w3 = jax.scipy.linalg.block_diag(*([w2] * n_chunks))

    nhr = n_chunks * _N_HASHES
    out_sds = [jax.ShapeDtypeStruct((n_groups, nhr, T), jnp.uint8)
               for _ in range(3)]
    r, g, b = pl.pallas_call(
        _make_hash_kernel(chg, nH, nW, ws, n_chunks),
        grid=(n_groups,),
        in_specs=[
            pl.BlockSpec((1, chg, nH, ws, nW, ws),
                         lambda n: (n, 0, 0, 0, 0, 0)),
            pl.BlockSpec((R, n_chunks * 16), lambda n: (0, 0)),
        ],
        out_specs=[pl.BlockSpec((1, nhr, T), lambda n: (n, 0, 0))
                   for _ in range(3)],
        out_shape=out_sds,
    )(x6, w3)

    def fin(a):
        a6 = a.reshape(B, _N_HASHES, nH, nW, ws, ws)
        return a6.transpose(0, 2, 4, 3, 5, 1).reshape(B, H, W, _N_HASHES)

    return fin(r), fin(g), fin(b)


# speedup vs baseline: 33.1662x; 1.0054x over previous
import math

import jax
import jax.numpy as jnp
from jax.experimental import pallas as pl

_WS = 32
_N_HASHES = 4
_HALF = 4

_COLOR_R = (0, 46, 167, 100, 191, 220, 0, 10)
_COLOR_G = (160, 141, 0, 62, 30, 87, 166, 91)
_COLOR_B = (177, 239, 174, 191, 75, 46, 0, 196)


def _make_hash_kernel(chg, nH, nW, ws, n_chunks):
    def _hash_kernel(x_ref, w_ref, r_ref, g_ref, b_ref):
        x = x_ref[0]
        xe = x.transpose(0, 1, 3, 2, 4).reshape(chg * nH * nW, ws, ws)
        w = w_ref[...]
        w2d = w.reshape(chg * nH * nW, -1)
        v = jax.lax.dot_general(w2d, xe, (((0,), (0,)), ((), ())),
                                preferred_element_type=jnp.float32)
        codes = []
        for q in range(n_chunks):
            for h in range(_N_HASHES):
                vh = v[q * 16 + h * _HALF:q * 16 + (h + 1) * _HALF]
                full = jnp.concatenate([vh, -vh], axis=0)
                codes.append(
                    jnp.argmax(full, axis=0, keepdims=True).astype(jnp.int32))
        code = jnp.concatenate(codes, axis=0)
        for ref, table in ((r_ref, _COLOR_R), (g_ref, _COLOR_G),
                           (b_ref, _COLOR_B)):
            acc = jnp.zeros(code.shape, jnp.int32)
            for k, val in enumerate(table):
                if val:
                    acc = acc + jnp.where(code == k, jnp.int32(val),
                                          jnp.int32(0))
            ref[0] = acc.astype(jnp.uint8)

    return _hash_kernel


def kernel(inp, rotations):
    B, C, H, W = inp.shape
    ws = _WS
    nH, nW = H // ws, W // ws
    M = nH * nW
    R = math.lcm(C, M)
    chg = R // M
    n_chunks = R // C
    n_groups = (B * C) // chg

    x6 = inp.reshape(n_groups, chg, nH, ws, nW, ws)
    w2 = rotations.reshape(C, _N_HASHES * _HALF)
    w3 = jax.scipy.linalg.block_diag(*([w2] * n_chunks))
    w3r = w3.reshape(chg * nH, nW, n_chunks * 16)

    nhr = n_chunks * _N_HASHES
    out_sds = [jax.ShapeDtypeStruct((n_groups, nhr, ws, ws), jnp.uint8)
               for _ in range(3)]
    r, g, b = pl.pallas_call(
        _make_hash_kernel(chg, nH, nW, ws, n_chunks),
        grid=(n_groups,),
        in_specs=[
            pl.BlockSpec((1, chg, nH, ws, nW, ws),
                         lambda n: (n, 0, 0, 0, 0, 0)),
            pl.BlockSpec((chg * nH, nW, n_chunks * 16), lambda n: (0, 0, 0)),
        ],
        out_specs=[pl.BlockSpec((1, nhr, ws, ws), lambda n: (n, 0, 0, 0))
                   for _ in range(3)],
        out_shape=out_sds,
    )(x6, w3r)

    def fin(a):
        a6 = a.reshape(B, _N_HASHES, nH, nW, ws, ws)
        return a6.transpose(0, 2, 4, 3, 5, 1).reshape(B, H, W, _N_HASHES)

    return fin(r), fin(g), fin(b)


# speedup vs baseline: 60.3101x; 1.8184x over previous
import math

import jax
import jax.numpy as jnp
from jax.experimental import pallas as pl

_WS = 32
_N_HASHES = 4
_HALF = 4

_COLOR_R = (0, 46, 167, 100, 191, 220, 0, 10)
_COLOR_G = (160, 141, 0, 62, 30, 87, 166, 91)
_COLOR_B = (177, 239, 174, 191, 75, 46, 0, 196)


def _make_hash_kernel(chg, nH, nW, ws, n_chunks):
    def _hash_kernel(x_ref, w_ref, r_ref, g_ref, b_ref):
        x = x_ref[0].reshape(chg, nH, ws, nW, ws)
        xe = x.transpose(0, 1, 3, 2, 4).reshape(chg * nH * nW, ws, ws)
        w = w_ref[...]
        w2d = w.reshape(chg * nH * nW, -1)
        v = jax.lax.dot_general(w2d, xe, (((0,), (0,)), ((), ())),
                                preferred_element_type=jnp.float32)
        codes = []
        for q in range(n_chunks):
            for h in range(_N_HASHES):
                vh = v[q * 16 + h * _HALF:q * 16 + (h + 1) * _HALF]
                full = jnp.concatenate([vh, -vh], axis=0)
                codes.append(
                    jnp.argmax(full, axis=0, keepdims=True).astype(jnp.int32))
        code = jnp.concatenate(codes, axis=0)
        for ref, table in ((r_ref, _COLOR_R), (g_ref, _COLOR_G),
                           (b_ref, _COLOR_B)):
            acc = jnp.zeros(code.shape, jnp.int32)
            for k, val in enumerate(table):
                if val:
                    acc = acc + jnp.where(code == k, jnp.int32(val),
                                          jnp.int32(0))
            ref[0] = acc.astype(jnp.uint8)

    return _hash_kernel


def kernel(inp, rotations):
    B, C, H, W = inp.shape
    ws = _WS
    nH, nW = H // ws, W // ws
    M = nH * nW
    R = math.lcm(C, M)
    chg = R // M
    n_chunks = R // C
    n_groups = (B * C) // chg

    x6 = inp.reshape(n_groups, chg, nH, ws, nW * ws)
    w2 = rotations.reshape(C, _N_HASHES * _HALF)
    w3 = jax.scipy.linalg.block_diag(*([w2] * n_chunks))
    w3r = w3.reshape(chg * nH, nW, n_chunks * 16)

    nhr = n_chunks * _N_HASHES
    out_sds = [jax.ShapeDtypeStruct((n_groups, nhr, ws, ws), jnp.uint8)
               for _ in range(3)]
    r, g, b = pl.pallas_call(
        _make_hash_kernel(chg, nH, nW, ws, n_chunks),
        grid=(n_groups,),
        in_specs=[
            pl.BlockSpec((1, chg, nH, ws, nW * ws),
                         lambda n: (n, 0, 0, 0, 0)),
            pl.BlockSpec((chg * nH, nW, n_chunks * 16), lambda n: (0, 0, 0)),
        ],
        out_specs=[pl.BlockSpec((1, nhr, ws, ws), lambda n: (n, 0, 0, 0))
                   for _ in range(3)],
        out_shape=out_sds,
    )(x6, w3r)

    def fin(a):
        a6 = a.reshape(B, _N_HASHES, nH, nW, ws, ws)
        return a6.transpose(0, 2, 4, 3, 5, 1).reshape(B, H, W, _N_HASHES)

    return fin(r), fin(g), fin(b)


# speedup vs baseline: 65.4837x; 1.0858x over previous
import math

import jax
import jax.numpy as jnp
from jax.experimental import pallas as pl

_WS = 32
_N_HASHES = 4
_HALF = 4

_COLOR_R = (0, 46, 167, 100, 191, 220, 0, 10)
_COLOR_G = (160, 141, 0, 62, 30, 87, 166, 91)
_COLOR_B = (177, 239, 174, 191, 75, 46, 0, 196)


def _make_hash_kernel(chg, nH, nW, ws, n_chunks):
    def _hash_kernel(x_ref, w_ref, r_ref, g_ref, b_ref):
        x = x_ref[0].reshape(chg * nH, ws, nW * ws)
        w = w_ref[...]
        v = None
        for j in range(nW):
            rhs = x[:, :, j * ws:(j + 1) * ws]
            pj = jax.lax.dot_general(w[j], rhs, (((0,), (0,)), ((), ())),
                                     preferred_element_type=jnp.float32)
            v = pj if v is None else v + pj
        codes = []
        for q in range(n_chunks):
            for h in range(_N_HASHES):
                vh = v[q * 16 + h * _HALF:q * 16 + (h + 1) * _HALF]
                full = jnp.concatenate([vh, -vh], axis=0)
                codes.append(
                    jnp.argmax(full, axis=0, keepdims=True).astype(jnp.int32))
        code = jnp.concatenate(codes, axis=0)
        for ref, table in ((r_ref, _COLOR_R), (g_ref, _COLOR_G),
                           (b_ref, _COLOR_B)):
            acc = jnp.zeros(code.shape, jnp.int32)
            for k, val in enumerate(table):
                if val:
                    acc = acc + jnp.where(code == k, jnp.int32(val),
                                          jnp.int32(0))
            ref[0] = acc.astype(jnp.uint8)

    return _hash_kernel


def kernel(inp, rotations):
    B, C, H, W = inp.shape
    ws = _WS
    nH, nW = H // ws, W // ws
    M = nH * nW
    R = math.lcm(C, M)
    chg = R // M
    n_chunks = R // C
    n_groups = (B * C) // chg

    x6 = inp.reshape(n_groups, chg, nH, ws, nW * ws)
    w2 = rotations.reshape(C, _N_HASHES * _HALF)
    w3 = jax.scipy.linalg.block_diag(*([w2] * n_chunks))
    w3r = (w3.reshape(chg, nH, nW, n_chunks * 16)
              .transpose(2, 0, 1, 3)
              .reshape(nW, chg * nH, n_chunks * 16))

    nhr = n_chunks * _N_HASHES
    out_sds = [jax.ShapeDtypeStruct((n_groups, nhr, ws, ws), jnp.uint8)
               for _ in range(3)]
    r, g, b = pl.pallas_call(
        _make_hash_kernel(chg, nH, nW, ws, n_chunks),
        grid=(n_groups,),
        in_specs=[
            pl.BlockSpec((1, chg, nH, ws, nW * ws),
                         lambda n: (n, 0, 0, 0, 0)),
            pl.BlockSpec((nW, chg * nH, n_chunks * 16), lambda n: (0, 0, 0)),
        ],
        out_specs=[pl.BlockSpec((1, nhr, ws, ws), lambda n: (n, 0, 0, 0))
                   for _ in range(3)],
        out_shape=out_sds,
    )(x6, w3r)

    def fin(a):
        a6 = a.reshape(B, _N_HASHES, nH, nW, ws, ws)
        return a6.transpose(0, 2, 4, 3, 5, 1).reshape(B, H, W, _N_HASHES)

    return fin(r), fin(g), fin(b)
